# baseline (speedup 1.0000x reference)
import jax
import jax.numpy as jnp
from jax.experimental import pallas as pl
from jax.experimental.pallas import tpu as pltpu

_BT = 1024


def _lora_body(slot_ref, x1_ref, x2_ref, a_ref, b_ref, out_ref, h_buf):
    j = pl.program_id(1)
    r = a_ref.shape[1] // 8
    dh = x1_ref.shape[1]
    dj = out_ref.shape[1]

    @pl.when(j == 0)
    def _():
        h = jnp.dot(x1_ref[...].astype(jnp.bfloat16), a_ref[:dh, :],
                    preferred_element_type=jnp.float32)
        h += jnp.dot(x2_ref[...].astype(jnp.bfloat16), a_ref[dh:, :],
                     preferred_element_type=jnp.float32)
        col_slot = jax.lax.broadcasted_iota(jnp.int32, h.shape, 1) // r
        mask = slot_ref[...] == col_slot
        h_buf[...] = jnp.where(mask, h, 0.0).astype(jnp.bfloat16)
        out_ref[...] = jnp.dot(h_buf[...], b_ref[:, :dj],
                               preferred_element_type=jnp.float32)

    @pl.when(j == 1)
    def _():
        out_ref[...] = jnp.dot(h_buf[...], b_ref[:, dj:],
                               preferred_element_type=jnp.float32)


def kernel(x, token_to_slot, lora_a, lora_b, lora_scaling):
    T, D = x.shape
    E, _, R = lora_a.shape
    Dout = lora_b.shape[-1]
    Dh = D // 2
    a_cat = jnp.transpose(lora_a, (1, 0, 2)).reshape(D, E * R).astype(jnp.bfloat16)
    b_cat = (lora_b * lora_scaling[:, None, None]).reshape(E * R, Dout)
    b_cat = b_cat.astype(jnp.bfloat16)
    slot2 = token_to_slot.reshape(T, 1)
    return pl.pallas_call(
        _lora_body,
        grid=(T // _BT, 2),
        in_specs=[
            pl.BlockSpec((_BT, 1), lambda i, j: (i, 0)),
            pl.BlockSpec((_BT, Dh), lambda i, j: (i, 0)),
            pl.BlockSpec((_BT, Dh), lambda i, j: (i, 1)),
            pl.BlockSpec((D, E * R), lambda i, j: (0, 0)),
            pl.BlockSpec((E * R, Dout), lambda i, j: (0, 0)),
        ],
        out_specs=pl.BlockSpec((_BT, Dout // 2), lambda i, j: (i, j)),
        out_shape=jax.ShapeDtypeStruct((T, Dout), x.dtype),
        scratch_shapes=[pltpu.VMEM((_BT, E * R), jnp.bfloat16)],
    )(slot2, x, x, a_cat, b_cat)


# speedup vs baseline: 1.1706x; 1.1706x over previous
import jax
import jax.numpy as jnp
from jax import lax
from jax.experimental import pallas as pl

_BTI = 2048
_BTO = 1024


def _lora_body(slot_ref, x_ref, a_ref, b_ref, out_ref):
    j = pl.program_id(1)
    r = a_ref.shape[1] // 8
    off = j * _BTO
    xs = x_ref[pl.ds(off, _BTO), :].astype(jnp.bfloat16)
    h = jnp.dot(xs, a_ref[...], preferred_element_type=jnp.float32)
    col_slot = jax.lax.broadcasted_iota(jnp.int32, h.shape, 1) // r
    mask = slot_ref[pl.ds(off, _BTO), :] == col_slot
    hb = jnp.where(mask, h, 0.0).astype(jnp.bfloat16)
    out_ref[...] = jnp.dot(hb, b_ref[...], preferred_element_type=jnp.float32)


def kernel(x, token_to_slot, lora_a, lora_b, lora_scaling):
    T, D = x.shape
    E, _, R = lora_a.shape
    Dout = lora_b.shape[-1]
    a_cat = jnp.transpose(lora_a, (1, 0, 2)).reshape(D, E * R).astype(jnp.bfloat16)
    b_cat = (lora_b * lora_scaling[:, None, None]).reshape(E * R, Dout).astype(jnp.bfloat16)
    slot2 = token_to_slot.reshape(T, 1)
    return pl.pallas_call(
        _lora_body,
        grid=(T // _BTI, 2),
        in_specs=[
            pl.BlockSpec((_BTI, 1), lambda i, j: (i, 0)),
            pl.BlockSpec((_BTI, D), lambda i, j: (i, 0)),
            pl.BlockSpec((D, E * R), lambda i, j: (0, 0)),
            pl.BlockSpec((E * R, Dout), lambda i, j: (0, 0)),
        ],
        out_specs=pl.BlockSpec((_BTO, Dout), lambda i, j: (2 * i + j, 0)),
        out_shape=jax.ShapeDtypeStruct((T, Dout), x.dtype),
    )(slot2, x, a_cat, b_cat)


# speedup vs baseline: 1.6234x; 1.3869x over previous
import jax
import jax.numpy as jnp
from jax.experimental import pallas as pl
from jax.experimental.pallas import tpu as pltpu

_BT = 1024


def _lora_body(slot_ref, x_ref, a_ref, b_ref, out_ref):
    r = a_ref.shape[1] // 8
    xb = x_ref[...].astype(jnp.bfloat16)
    h = jnp.dot(xb, a_ref[...], preferred_element_type=jnp.float32)
    col_slot = jax.lax.broadcasted_iota(jnp.int32, h.shape, 1) // r
    mask = slot_ref[...] == col_slot
    hb = jnp.where(mask, h, 0.0).astype(jnp.bfloat16)
    out_ref[...] = jnp.dot(hb, b_ref[...], preferred_element_type=jnp.float32)


def kernel(x, token_to_slot, lora_a, lora_b, lora_scaling):
    T, D = x.shape
    E, _, R = lora_a.shape
    Dout = lora_b.shape[-1]
    a_cat = jnp.transpose(lora_a, (1, 0, 2)).reshape(D, E * R)
    b_cat = (lora_b * lora_scaling[:, None, None]).reshape(E * R, Dout)
    a_cat = a_cat.astype(jnp.bfloat16)
    b_cat = b_cat.astype(jnp.bfloat16)
    slot2 = token_to_slot.reshape(T, 1)
    return pl.pallas_call(
        _lora_body,
        grid=(T // _BT,),
        in_specs=[
            pl.BlockSpec((_BT, 1), lambda i: (i, 0)),
            pl.BlockSpec((_BT, D), lambda i: (i, 0)),
            pl.BlockSpec((D, E * R), lambda i: (0, 0)),
            pl.BlockSpec((E * R, Dout), lambda i: (0, 0)),
        ],
        out_specs=pl.BlockSpec((_BT, Dout), lambda i: (i, 0)),
        out_shape=jax.ShapeDtypeStruct((T, Dout), x.dtype),
    )(slot2, x, a_cat, b_cat)


# speedup vs baseline: 1.6257x; 1.0014x over previous
import jax
import jax.numpy as jnp
from jax.experimental import pallas as pl
from jax.experimental.pallas import tpu as pltpu

_BT = 1024


def _lora_body(slot_ref, x_ref, a_ref, b_ref, out_ref):
    r = a_ref.shape[1] // 8
    xb = x_ref[...].astype(jnp.bfloat16)
    h = jnp.dot(xb, a_ref[...], preferred_element_type=jnp.float32)
    col_slot = jax.lax.broadcasted_iota(jnp.int32, h.shape, 1) // r
    mask = slot_ref[...] == col_slot
    hb = jnp.where(mask, h, 0.0).astype(jnp.bfloat16)
    out_ref[...] = jnp.dot(hb, b_ref[...], preferred_element_type=jnp.float32)


def kernel(x, token_to_slot, lora_a, lora_b, lora_scaling):
    T, D = x.shape
    E, _, R = lora_a.shape
    Dout = lora_b.shape[-1]
    a_cat = jnp.transpose(lora_a, (1, 0, 2)).reshape(D, E * R)
    b_cat = (lora_b * lora_scaling[:, None, None]).reshape(E * R, Dout)
    a_cat = a_cat.astype(jnp.bfloat16)
    b_cat = b_cat.astype(jnp.bfloat16)
    slot2 = token_to_slot.reshape(T, 1)
    return pl.pallas_call(
        _lora_body,
        grid=(T // _BT,),
        in_specs=[
            pl.BlockSpec((_BT, 1), lambda i: (i, 0)),
            pl.BlockSpec((_BT, D), lambda i: (i, 0)),
            pl.BlockSpec((D, E * R), lambda i: (0, 0)),
            pl.BlockSpec((E * R, Dout), lambda i: (0, 0)),
        ],
        out_specs=pl.BlockSpec((_BT, Dout), lambda i: (i, 0)),
        out_shape=jax.ShapeDtypeStruct((T, Dout), x.dtype),
        compiler_params=pltpu.CompilerParams(
            dimension_semantics=("parallel",)),
    )(slot2, x, a_cat, b_cat)
